# initial kernel scaffold (unmeasured)
import jax
import jax.numpy as jnp
from jax import lax
from jax.experimental import pallas as pl
from jax.experimental.pallas import tpu as pltpu

M_CHUNK = 768


def kernel(A, B):
    M, K = A.shape
    K2, N = B.shape
    assert K == K2
    n_chunks = M // M_CHUNK

    def body(a_ref, b_ref, out_ref, b_bf_ref, recv_ref, send_sem, recv_sem):
        my_x = lax.axis_index("x")
        my_y = lax.axis_index("y")
        peer = (1 - my_x, my_y)

        b_bf_ref[...] = b_ref[...].astype(jnp.bfloat16)

        for c in range(n_chunks):
            rows = pl.ds(c * M_CHUNK, M_CHUNK)
            a_chunk = a_ref[rows, :].astype(jnp.bfloat16)
            out_ref[rows, :] = jnp.dot(
                a_chunk, b_bf_ref[...], preferred_element_type=jnp.float32
            ).astype(jnp.bfloat16)

        barrier_sem = pltpu.get_barrier_semaphore()
        pl.semaphore_signal(
            barrier_sem, inc=1, device_id=peer,
            device_id_type=pl.DeviceIdType.MESH,
        )
        pl.semaphore_wait(barrier_sem, 1)

        rdma = pltpu.make_async_remote_copy(
            src_ref=out_ref,
            dst_ref=recv_ref,
            send_sem=send_sem,
            recv_sem=recv_sem,
            device_id=peer,
            device_id_type=pl.DeviceIdType.MESH,
        )
        rdma.start()
        rdma.wait()

        for c in range(n_chunks):
            rows = pl.ds(c * M_CHUNK, M_CHUNK)
            out_ref[rows, :] = (
                out_ref[rows, :].astype(jnp.float32)
                + recv_ref[rows, :].astype(jnp.float32)
            ).astype(jnp.bfloat16)

    return pl.pallas_call(
        body,
        out_shape=jax.ShapeDtypeStruct((M, N), jnp.bfloat16),
        in_specs=[
            pl.BlockSpec(memory_space=pltpu.VMEM),
            pl.BlockSpec(memory_space=pltpu.VMEM),
        ],
        out_specs=pl.BlockSpec(memory_space=pltpu.VMEM),
        scratch_shapes=[
            pltpu.VMEM((K, N), jnp.bfloat16),
            pltpu.VMEM((M, N), jnp.bfloat16),
            pltpu.SemaphoreType.DMA,
            pltpu.SemaphoreType.DMA,
        ],
        compiler_params=pltpu.CompilerParams(collective_id=0),
    )(A, B)


# baseline (device time: 286154 ns/iter reference)
import jax
import jax.numpy as jnp
from jax import lax
from jax.experimental import pallas as pl
from jax.experimental.pallas import tpu as pltpu

M_CHUNK = 384


def kernel(A, B):
    M, K = A.shape
    K2, N = B.shape
    assert K == K2
    n_chunks = M // M_CHUNK

    A = A.astype(jnp.bfloat16)
    B = B.astype(jnp.bfloat16)

    def body(a_ref, b_ref, out_ref, recv_ref, send_sem, recv_sem):
        my_x = lax.axis_index("x")
        my_y = lax.axis_index("y")
        peer = (1 - my_x, my_y)

        for c in range(n_chunks):
            rows = pl.ds(c * M_CHUNK, M_CHUNK)
            out_ref[rows, :] = jnp.dot(
                a_ref[rows, :], b_ref[...], preferred_element_type=jnp.float32
            ).astype(jnp.bfloat16)

        barrier_sem = pltpu.get_barrier_semaphore()
        pl.semaphore_signal(
            barrier_sem, inc=1, device_id=peer,
            device_id_type=pl.DeviceIdType.MESH,
        )
        pl.semaphore_wait(barrier_sem, 1)

        rdma = pltpu.make_async_remote_copy(
            src_ref=out_ref,
            dst_ref=recv_ref,
            send_sem=send_sem,
            recv_sem=recv_sem,
            device_id=peer,
            device_id_type=pl.DeviceIdType.MESH,
        )
        rdma.start()
        rdma.wait()

        for c in range(n_chunks):
            rows = pl.ds(c * M_CHUNK, M_CHUNK)
            out_ref[rows, :] = (
                out_ref[rows, :].astype(jnp.float32)
                + recv_ref[rows, :].astype(jnp.float32)
            ).astype(jnp.bfloat16)

    return pl.pallas_call(
        body,
        out_shape=jax.ShapeDtypeStruct((M, N), jnp.bfloat16),
        in_specs=[
            pl.BlockSpec(memory_space=pltpu.VMEM),
            pl.BlockSpec(memory_space=pltpu.VMEM),
        ],
        out_specs=pl.BlockSpec(memory_space=pltpu.VMEM),
        scratch_shapes=[
            pltpu.VMEM((M, N), jnp.bfloat16),
            pltpu.SemaphoreType.DMA,
            pltpu.SemaphoreType.DMA,
        ],
        compiler_params=pltpu.CompilerParams(
            collective_id=0,
            vmem_limit_bytes=60 * 1024 * 1024,
        ),
    )(A, B)


# device time: 255533 ns/iter; 1.1198x vs baseline; 1.1198x over previous
import jax
import jax.numpy as jnp
from jax import lax
from jax.experimental import pallas as pl
from jax.experimental.pallas import tpu as pltpu

M_CHUNK = 384


def kernel(A, B):
    M, K = A.shape
    K2, N = B.shape
    assert K == K2
    n_chunks = M // M_CHUNK

    A = A.astype(jnp.bfloat16)
    B = B.astype(jnp.bfloat16)

    def body(a_ref, b_ref, out_ref, recv_ref, send_sems, recv_sems):
        my_x = lax.axis_index("x")
        my_y = lax.axis_index("y")
        peer = (1 - my_x, my_y)

        barrier_sem = pltpu.get_barrier_semaphore()
        pl.semaphore_signal(
            barrier_sem, inc=1, device_id=peer,
            device_id_type=pl.DeviceIdType.MESH,
        )
        pl.semaphore_wait(barrier_sem, 1)

        rdmas = []
        for c in range(n_chunks):
            rows = pl.ds(c * M_CHUNK, M_CHUNK)
            out_ref[rows, :] = jnp.dot(
                a_ref[rows, :], b_ref[...], preferred_element_type=jnp.float32
            ).astype(jnp.bfloat16)
            rdma = pltpu.make_async_remote_copy(
                src_ref=out_ref.at[rows, :],
                dst_ref=recv_ref.at[rows, :],
                send_sem=send_sems.at[c],
                recv_sem=recv_sems.at[c],
                device_id=peer,
                device_id_type=pl.DeviceIdType.MESH,
            )
            rdma.start()
            rdmas.append(rdma)

        for c in range(n_chunks):
            rows = pl.ds(c * M_CHUNK, M_CHUNK)
            rdmas[c].wait()
            out_ref[rows, :] = (
                out_ref[rows, :].astype(jnp.float32)
                + recv_ref[rows, :].astype(jnp.float32)
            ).astype(jnp.bfloat16)

    return pl.pallas_call(
        body,
        out_shape=jax.ShapeDtypeStruct((M, N), jnp.bfloat16),
        in_specs=[
            pl.BlockSpec(memory_space=pltpu.VMEM),
            pl.BlockSpec(memory_space=pltpu.VMEM),
        ],
        out_specs=pl.BlockSpec(memory_space=pltpu.VMEM),
        scratch_shapes=[
            pltpu.VMEM((M, N), jnp.bfloat16),
            pltpu.SemaphoreType.DMA((n_chunks,)),
            pltpu.SemaphoreType.DMA((n_chunks,)),
        ],
        compiler_params=pltpu.CompilerParams(
            collective_id=0,
            vmem_limit_bytes=60 * 1024 * 1024,
        ),
    )(A, B)


# device time: 210037 ns/iter; 1.3624x vs baseline; 1.2166x over previous
import jax
import jax.numpy as jnp
from jax import lax
from jax.experimental import pallas as pl
from jax.experimental.pallas import tpu as pltpu

M_CHUNK = 384
N_A_SUB = 2
N_B_CHUNK = 4


def kernel(A, B):
    M, K = A.shape
    K2, N = B.shape
    assert K == K2
    half_m = M // 2
    sub_m = half_m // N_A_SUB
    cb = N // N_B_CHUNK
    n_chunks = M // M_CHUNK

    A = A.astype(jnp.bfloat16)
    B = B.astype(jnp.bfloat16)

    def body(a_ref, b_ref, out_ref, a_other, b_other,
             ax_send, ax_recv, ay_send, ay_recv, b_send, b_recv):
        my_x = lax.axis_index("x")
        my_y = lax.axis_index("y")
        x_peer = (1 - my_x, my_y)
        y_peer = (my_x, 1 - my_y)

        barrier_sem = pltpu.get_barrier_semaphore()
        for nbr in (x_peer, y_peer):
            pl.semaphore_signal(
                barrier_sem, inc=1, device_id=nbr,
                device_id_type=pl.DeviceIdType.MESH,
            )
        pl.semaphore_wait(barrier_sem, 2)

        a_x = []
        for s in range(N_A_SUB):
            rows = pl.ds(my_y * half_m + s * sub_m, sub_m)
            r = pltpu.make_async_remote_copy(
                src_ref=a_ref.at[rows, :],
                dst_ref=a_other.at[rows, :],
                send_sem=ax_send.at[s],
                recv_sem=ax_recv.at[s],
                device_id=x_peer,
                device_id_type=pl.DeviceIdType.MESH,
            )
            r.start()
            a_x.append(r)
        b_x = []
        for c in range(N_B_CHUNK):
            cols = pl.ds(c * cb, cb)
            r = pltpu.make_async_remote_copy(
                src_ref=b_ref.at[:, cols],
                dst_ref=b_other.at[:, cols],
                send_sem=b_send.at[c],
                recv_sem=b_recv.at[c],
                device_id=x_peer,
                device_id_type=pl.DeviceIdType.MESH,
            )
            r.start()
            b_x.append(r)

        fwd = []

        def forward(s):
            a_x[s].wait_recv()
            rows = pl.ds(my_y * half_m + s * sub_m, sub_m)
            r = pltpu.make_async_remote_copy(
                src_ref=a_other.at[rows, :],
                dst_ref=a_other.at[rows, :],
                send_sem=ay_send.at[s],
                recv_sem=ay_recv.at[s],
                device_id=y_peer,
                device_id_type=pl.DeviceIdType.MESH,
            )
            r.start()
            fwd.append(r)

        for c in range(n_chunks):
            rows = pl.ds(c * M_CHUNK, M_CHUNK)
            out_ref[rows, :] = jnp.dot(
                a_ref[rows, :], b_ref[...], preferred_element_type=jnp.float32
            ).astype(jnp.bfloat16)
            if c == 2:
                forward(0)
            if c == 4:
                forward(1)

        for s in range(N_A_SUB):
            rows = pl.ds((1 - my_y) * half_m + s * sub_m, sub_m)
            recv = pltpu.make_async_remote_copy(
                src_ref=a_other.at[rows, :],
                dst_ref=a_other.at[rows, :],
                send_sem=ay_send.at[s],
                recv_sem=ay_recv.at[s],
                device_id=y_peer,
                device_id_type=pl.DeviceIdType.MESH,
            )
            recv.wait_recv()

        for c in range(N_B_CHUNK):
            b_x[c].wait_recv()
            cols = pl.ds(c * cb, cb)
            for h in range(2):
                rows = pl.ds(h * half_m, half_m)
                out_ref[rows, cols] = (
                    out_ref[rows, cols].astype(jnp.float32)
                    + jnp.dot(
                        a_other[rows, :], b_other[:, cols],
                        preferred_element_type=jnp.float32,
                    )
                ).astype(jnp.bfloat16)

        for r in a_x + b_x + fwd:
            r.wait_send()

    return pl.pallas_call(
        body,
        out_shape=jax.ShapeDtypeStruct((M, N), jnp.bfloat16),
        in_specs=[
            pl.BlockSpec(memory_space=pltpu.VMEM),
            pl.BlockSpec(memory_space=pltpu.VMEM),
        ],
        out_specs=pl.BlockSpec(memory_space=pltpu.VMEM),
        scratch_shapes=[
            pltpu.VMEM((M, K), jnp.bfloat16),
            pltpu.VMEM((K, N), jnp.bfloat16),
            pltpu.SemaphoreType.DMA((N_A_SUB,)),
            pltpu.SemaphoreType.DMA((N_A_SUB,)),
            pltpu.SemaphoreType.DMA((N_A_SUB,)),
            pltpu.SemaphoreType.DMA((N_A_SUB,)),
            pltpu.SemaphoreType.DMA((N_B_CHUNK,)),
            pltpu.SemaphoreType.DMA((N_B_CHUNK,)),
        ],
        compiler_params=pltpu.CompilerParams(
            collective_id=0,
            vmem_limit_bytes=61 * 1024 * 1024,
        ),
    )(A, B)
